# baseline (speedup 1.0000x reference)
import functools

import jax
import jax.numpy as jnp
from jax import lax
from jax.experimental import pallas as pl
from jax.experimental.pallas import tpu as pltpu
from jax.experimental.pallas import tpu_sc as plsc

M = 8192
N = 8192
K = 256
COMMIT = 0.25


BM = 1024
BN = 512
NB_M = M // BM
NB_N = N // BN
_WINDOW_BLOCKS = 4096 // BN


def _mm_argmax_body(z_ref, w_ref, idx_ref, rmax_ref, ridx_ref, zn_ref, wn_ref):
    m = pl.program_id(0)
    n = pl.program_id(1)

    @pl.when(m == 0)
    def _():
        w = w_ref[pl.ds(n * BN, BN), :]
        s = jnp.sum(w * w, axis=1, keepdims=True)
        wn_ref[pl.ds(n * BN, BN), :] = w / jnp.maximum(jnp.sqrt(s), 1e-12)

    @pl.when(n == 0)
    def _():
        z = z_ref[...]
        zs = jnp.sum(z * z, axis=1, keepdims=True)
        zn_ref[...] = z / jnp.maximum(jnp.sqrt(zs), 1e-12)

    sim = lax.dot_general(zn_ref[...], wn_ref[pl.ds(n * BN, BN), :],
                          (((1,), (1,)), ((), ())),
                          preferred_element_type=jnp.float32)

    bmax = jnp.max(sim, axis=1, keepdims=True)
    iota = lax.broadcasted_iota(jnp.int32, (BM, BN), 1)
    bidx = jnp.min(jnp.where(sim == bmax, iota, jnp.int32(2**30)),
                   axis=1, keepdims=True) + n * BN

    @pl.when(n == 0)
    def _():
        rmax_ref[...] = jnp.full((BM, 1), -jnp.inf, jnp.float32)
        ridx_ref[...] = jnp.zeros((BM, 1), jnp.int32)

    better = bmax > rmax_ref[...]
    ridx_ref[...] = jnp.where(better, bidx, ridx_ref[...])
    rmax_ref[...] = jnp.where(better, bmax, rmax_ref[...])

    @pl.when(jnp.logical_and((n + 1) % _WINDOW_BLOCKS == 0, n != NB_N - 1))
    def _():
        rmax_ref[...] = rmax_ref[...].astype(jnp.bfloat16).astype(jnp.float32)

    @pl.when(n == NB_N - 1)
    def _():
        idx_ref[...] = ridx_ref[...]


def _mm_argmax(z, W, interpret=False):
    return pl.pallas_call(
        _mm_argmax_body,
        grid=(NB_M, NB_N),
        in_specs=[pl.BlockSpec((BM, K), lambda m, n: (m, 0)),
                  pl.BlockSpec((N, K), lambda m, n: (0, 0))],
        out_specs=pl.BlockSpec((BM, 1), lambda m, n: (m, 0)),
        out_shape=jax.ShapeDtypeStruct((M, 1), jnp.int32),
        scratch_shapes=[pltpu.VMEM((BM, 1), jnp.float32),
                        pltpu.VMEM((BM, 1), jnp.int32),
                        pltpu.VMEM((BM, K), jnp.float32),
                        pltpu.VMEM((N, K), jnp.float32)],
        compiler_params=pltpu.CompilerParams(
            dimension_semantics=("arbitrary", "arbitrary")),
        interpret=interpret,
    )(z, W)


NC = 2
NS = 16
NW = NC * NS
BPW = M // NW

@functools.cache
def _sc_gather_kernel():
    mesh = plsc.VectorSubcoreMesh(core_axis_name="c", subcore_axis_name="s")

    @functools.partial(
        pl.kernel,
        out_type=jax.ShapeDtypeStruct((M, K), jnp.float32),
        mesh=mesh,
        scratch_types=[pltpu.VMEM((BPW,), jnp.int32),
                       pltpu.VMEM((BPW, K), jnp.float32),
                       pltpu.SemaphoreType.DMA],
    )
    def _sc_gather(idx_hbm, w_hbm, out_hbm, idx_v, rows_v, sem):
        wid = lax.axis_index("s") * NC + lax.axis_index("c")
        base = wid * BPW
        pltpu.sync_copy(idx_hbm.at[pl.ds(base, BPW)], idx_v)
        pltpu.async_copy(w_hbm.at[idx_v], rows_v, sem).wait()
        pltpu.sync_copy(rows_v, out_hbm.at[pl.ds(base, BPW)])

    return _sc_gather


BL = 1024
NB_L = M // BL


def _st_loss_body(z_ref, zq_ref, out_ref, loss_ref, acc_ref):
    i = pl.program_id(0)

    @pl.when(i == 0)
    def _():
        acc_ref[0] = 0.0

    z = z_ref[...]
    d = zq_ref[...] - z
    out_ref[...] = z + d
    acc_ref[0] += jnp.sum(d * d)

    @pl.when(i == NB_L - 1)
    def _():
        loss_ref[0, 0] = acc_ref[0] * (COMMIT / (M * K))


def _st_loss(z, zq, interpret=False):
    return pl.pallas_call(
        _st_loss_body,
        grid=(NB_L,),
        in_specs=[pl.BlockSpec((BL, K), lambda i: (i, 0)),
                  pl.BlockSpec((BL, K), lambda i: (i, 0))],
        out_specs=[pl.BlockSpec((BL, K), lambda i: (i, 0)),
                   pl.BlockSpec((1, 1), lambda i: (0, 0),
                                memory_space=pltpu.SMEM)],
        out_shape=[jax.ShapeDtypeStruct((M, K), jnp.float32),
                   jax.ShapeDtypeStruct((1, 1), jnp.float32)],
        scratch_shapes=[pltpu.SMEM((1,), jnp.float32)],
        compiler_params=pltpu.CompilerParams(
            dimension_semantics=("arbitrary",)),
        interpret=interpret,
    )(z, zq)


def kernel(z, W):
    idx = _mm_argmax(z, W).reshape(M)
    zq = _sc_gather_kernel()(idx, W)
    zq_st, loss = _st_loss(z, zq)
    return zq_st, loss.reshape(())


# speedup vs baseline: 1.2440x; 1.2440x over previous
import functools

import jax
import jax.numpy as jnp
from jax import lax
from jax.experimental import pallas as pl
from jax.experimental.pallas import tpu as pltpu
from jax.experimental.pallas import tpu_sc as plsc

M = 8192
N = 8192
K = 256
COMMIT = 0.25


BM = 1024
BN = 1024
NB_M = M // BM
NB_N = N // BN
_WINDOW_BLOCKS = 4096 // BN


def _mm_argmax_body(z_ref, w_ref, idx_ref, rmax_ref, ridx_ref, zn_ref, wn_ref):
    m = pl.program_id(0)
    n = pl.program_id(1)

    @pl.when(m == 0)
    def _():
        w = w_ref[pl.ds(n * BN, BN), :]
        s = jnp.sum(w * w, axis=1, keepdims=True)
        wn_ref[pl.ds(n * BN, BN), :] = w / jnp.maximum(jnp.sqrt(s), 1e-12)

    @pl.when(n == 0)
    def _():
        z = z_ref[...]
        zs = jnp.sum(z * z, axis=1, keepdims=True)
        zn_ref[...] = z / jnp.maximum(jnp.sqrt(zs), 1e-12)

    sim = lax.dot_general(zn_ref[...], wn_ref[pl.ds(n * BN, BN), :],
                          (((1,), (1,)), ((), ())),
                          preferred_element_type=jnp.float32)

    bmax = jnp.max(sim, axis=1, keepdims=True)
    iota = lax.broadcasted_iota(jnp.int32, (BM, BN), 1)
    bidx = jnp.min(jnp.where(sim == bmax, iota, jnp.int32(2**30)),
                   axis=1, keepdims=True) + n * BN

    @pl.when(n == 0)
    def _():
        rmax_ref[...] = jnp.full((BM, 1), -jnp.inf, jnp.float32)
        ridx_ref[...] = jnp.zeros((BM, 1), jnp.int32)

    better = bmax > rmax_ref[...]
    ridx_ref[...] = jnp.where(better, bidx, ridx_ref[...])
    rmax_ref[...] = jnp.where(better, bmax, rmax_ref[...])

    @pl.when(jnp.logical_and((n + 1) % _WINDOW_BLOCKS == 0, n != NB_N - 1))
    def _():
        rmax_ref[...] = rmax_ref[...].astype(jnp.bfloat16).astype(jnp.float32)

    @pl.when(n == NB_N - 1)
    def _():
        idx_ref[...] = ridx_ref[...]


def _mm_argmax(z, W, interpret=False):
    return pl.pallas_call(
        _mm_argmax_body,
        grid=(NB_M, NB_N),
        in_specs=[pl.BlockSpec((BM, K), lambda m, n: (m, 0)),
                  pl.BlockSpec((N, K), lambda m, n: (0, 0))],
        out_specs=pl.BlockSpec((BM, 1), lambda m, n: (m, 0)),
        out_shape=jax.ShapeDtypeStruct((M, 1), jnp.int32),
        scratch_shapes=[pltpu.VMEM((BM, 1), jnp.float32),
                        pltpu.VMEM((BM, 1), jnp.int32),
                        pltpu.VMEM((BM, K), jnp.float32),
                        pltpu.VMEM((N, K), jnp.float32)],
        compiler_params=pltpu.CompilerParams(
            dimension_semantics=("arbitrary", "arbitrary")),
        interpret=interpret,
    )(z, W)


NC = 2
NS = 16
NW = NC * NS
BPW = M // NW

@functools.cache
def _sc_gather_kernel():
    mesh = plsc.VectorSubcoreMesh(core_axis_name="c", subcore_axis_name="s")

    @functools.partial(
        pl.kernel,
        out_type=jax.ShapeDtypeStruct((M, K), jnp.float32),
        mesh=mesh,
        scratch_types=[pltpu.VMEM((BPW,), jnp.int32),
                       pltpu.VMEM((BPW, K), jnp.float32),
                       pltpu.SemaphoreType.DMA],
    )
    def _sc_gather(idx_hbm, w_hbm, out_hbm, idx_v, rows_v, sem):
        wid = lax.axis_index("s") * NC + lax.axis_index("c")
        base = wid * BPW
        pltpu.sync_copy(idx_hbm.at[pl.ds(base, BPW)], idx_v)
        pltpu.async_copy(w_hbm.at[idx_v], rows_v, sem).wait()
        pltpu.sync_copy(rows_v, out_hbm.at[pl.ds(base, BPW)])

    return _sc_gather


BL = 1024
NB_L = M // BL


def _st_loss_body(z_ref, zq_ref, out_ref, loss_ref, acc_ref):
    i = pl.program_id(0)

    @pl.when(i == 0)
    def _():
        acc_ref[0] = 0.0

    z = z_ref[...]
    d = zq_ref[...] - z
    out_ref[...] = z + d
    acc_ref[0] += jnp.sum(d * d)

    @pl.when(i == NB_L - 1)
    def _():
        loss_ref[0, 0] = acc_ref[0] * (COMMIT / (M * K))


def _st_loss(z, zq, interpret=False):
    return pl.pallas_call(
        _st_loss_body,
        grid=(NB_L,),
        in_specs=[pl.BlockSpec((BL, K), lambda i: (i, 0)),
                  pl.BlockSpec((BL, K), lambda i: (i, 0))],
        out_specs=[pl.BlockSpec((BL, K), lambda i: (i, 0)),
                   pl.BlockSpec((1, 1), lambda i: (0, 0),
                                memory_space=pltpu.SMEM)],
        out_shape=[jax.ShapeDtypeStruct((M, K), jnp.float32),
                   jax.ShapeDtypeStruct((1, 1), jnp.float32)],
        scratch_shapes=[pltpu.SMEM((1,), jnp.float32)],
        compiler_params=pltpu.CompilerParams(
            dimension_semantics=("arbitrary",)),
        interpret=interpret,
    )(z, zq)


def kernel(z, W):
    idx = _mm_argmax(z, W).reshape(M)
    zq = _sc_gather_kernel()(idx, W)
    zq_st, loss = _st_loss(z, zq)
    return zq_st, loss.reshape(())


# speedup vs baseline: 1.3346x; 1.0728x over previous
import functools

import jax
import jax.numpy as jnp
from jax import lax
from jax.experimental import pallas as pl
from jax.experimental.pallas import tpu as pltpu
from jax.experimental.pallas import tpu_sc as plsc

M = 8192
N = 8192
K = 256
COMMIT = 0.25


BM = 2048
BN = 1024
NB_M = M // BM
NB_N = N // BN
_WINDOW_BLOCKS = 4096 // BN


def _mm_argmax_body(z_ref, w_ref, idx_ref, rmax_ref, ridx_ref, zn_ref, wn_ref):
    m = pl.program_id(0)
    n = pl.program_id(1)

    @pl.when(m == 0)
    def _():
        w = w_ref[pl.ds(n * BN, BN), :]
        s = jnp.sum(w * w, axis=1, keepdims=True)
        wn_ref[pl.ds(n * BN, BN), :] = w / jnp.maximum(jnp.sqrt(s), 1e-12)

    @pl.when(n == 0)
    def _():
        z = z_ref[...]
        zs = jnp.sum(z * z, axis=1, keepdims=True)
        zn_ref[...] = z / jnp.maximum(jnp.sqrt(zs), 1e-12)

    sim = lax.dot_general(zn_ref[...], wn_ref[pl.ds(n * BN, BN), :],
                          (((1,), (1,)), ((), ())),
                          preferred_element_type=jnp.float32)

    bmax = jnp.max(sim, axis=1, keepdims=True)
    iota = lax.broadcasted_iota(jnp.int32, (BM, BN), 1)
    bidx = jnp.min(jnp.where(sim == bmax, iota, jnp.int32(2**30)),
                   axis=1, keepdims=True) + n * BN

    @pl.when(n == 0)
    def _():
        rmax_ref[...] = jnp.full((BM, 1), -jnp.inf, jnp.float32)
        ridx_ref[...] = jnp.zeros((BM, 1), jnp.int32)

    better = bmax > rmax_ref[...]
    ridx_ref[...] = jnp.where(better, bidx, ridx_ref[...])
    rmax_ref[...] = jnp.where(better, bmax, rmax_ref[...])

    @pl.when(jnp.logical_and((n + 1) % _WINDOW_BLOCKS == 0, n != NB_N - 1))
    def _():
        rmax_ref[...] = rmax_ref[...].astype(jnp.bfloat16).astype(jnp.float32)

    @pl.when(n == NB_N - 1)
    def _():
        idx_ref[...] = ridx_ref[...]


def _mm_argmax(z, W, interpret=False):
    return pl.pallas_call(
        _mm_argmax_body,
        grid=(NB_M, NB_N),
        in_specs=[pl.BlockSpec((BM, K), lambda m, n: (m, 0)),
                  pl.BlockSpec((N, K), lambda m, n: (0, 0))],
        out_specs=pl.BlockSpec((BM, 1), lambda m, n: (m, 0)),
        out_shape=jax.ShapeDtypeStruct((M, 1), jnp.int32),
        scratch_shapes=[pltpu.VMEM((BM, 1), jnp.float32),
                        pltpu.VMEM((BM, 1), jnp.int32),
                        pltpu.VMEM((BM, K), jnp.float32),
                        pltpu.VMEM((N, K), jnp.float32)],
        compiler_params=pltpu.CompilerParams(
            dimension_semantics=("arbitrary", "arbitrary")),
        interpret=interpret,
    )(z, W)


NC = 2
NS = 16
NW = NC * NS
BPW = M // NW

@functools.cache
def _sc_gather_kernel():
    mesh = plsc.VectorSubcoreMesh(core_axis_name="c", subcore_axis_name="s")

    @functools.partial(
        pl.kernel,
        out_type=jax.ShapeDtypeStruct((M, K), jnp.float32),
        mesh=mesh,
        scratch_types=[pltpu.VMEM((BPW,), jnp.int32),
                       pltpu.VMEM((BPW, K), jnp.float32),
                       pltpu.SemaphoreType.DMA],
    )
    def _sc_gather(idx_hbm, w_hbm, out_hbm, idx_v, rows_v, sem):
        wid = lax.axis_index("s") * NC + lax.axis_index("c")
        base = wid * BPW
        pltpu.sync_copy(idx_hbm.at[pl.ds(base, BPW)], idx_v)
        pltpu.async_copy(w_hbm.at[idx_v], rows_v, sem).wait()
        pltpu.sync_copy(rows_v, out_hbm.at[pl.ds(base, BPW)])

    return _sc_gather


BL = 1024
NB_L = M // BL


def _st_loss_body(z_ref, zq_ref, out_ref, loss_ref, acc_ref):
    i = pl.program_id(0)

    @pl.when(i == 0)
    def _():
        acc_ref[0] = 0.0

    z = z_ref[...]
    d = zq_ref[...] - z
    out_ref[...] = z + d
    acc_ref[0] += jnp.sum(d * d)

    @pl.when(i == NB_L - 1)
    def _():
        loss_ref[0, 0] = acc_ref[0] * (COMMIT / (M * K))


def _st_loss(z, zq, interpret=False):
    return pl.pallas_call(
        _st_loss_body,
        grid=(NB_L,),
        in_specs=[pl.BlockSpec((BL, K), lambda i: (i, 0)),
                  pl.BlockSpec((BL, K), lambda i: (i, 0))],
        out_specs=[pl.BlockSpec((BL, K), lambda i: (i, 0)),
                   pl.BlockSpec((1, 1), lambda i: (0, 0),
                                memory_space=pltpu.SMEM)],
        out_shape=[jax.ShapeDtypeStruct((M, K), jnp.float32),
                   jax.ShapeDtypeStruct((1, 1), jnp.float32)],
        scratch_shapes=[pltpu.SMEM((1,), jnp.float32)],
        compiler_params=pltpu.CompilerParams(
            dimension_semantics=("arbitrary",)),
        interpret=interpret,
    )(z, zq)


def kernel(z, W):
    idx = _mm_argmax(z, W).reshape(M)
    zq = _sc_gather_kernel()(idx, W)
    zq_st, loss = _st_loss(z, zq)
    return zq_st, loss.reshape(())
